# baseline (speedup 1.0000x reference)
import functools
import jax
import jax.numpy as jnp
from jax import lax
from jax.experimental import pallas as pl
from jax.experimental.pallas import tpu as pltpu
from jax.experimental.pallas import tpu_sc as plsc

_N_CAT = 26
_N_CONT = 13
_VOCAB = 100000
_DIM = 32
_B = 16384

_NC = 2
_NS = 16
_NW = _NC * _NS
_V = _VOCAB + 1
_BC = 4096
_NBC = _B // _BC
_L = 16
_FH = _N_CAT // 2


def _gather_chunk(row_v, col_v, ov, c):
    @plsc.parallel_loop(0, _BC, _L, unroll=16)
    def _(i):
        v = col_v[pl.ds(c * _BC + i, _L)].astype(jnp.int32)
        ov[pl.ds(i, _L)] = plsc.load_gather(row_v, [v])


def _emb_body(tabs_hbm, xt_hbm, out_hbm,
              row_v, col_v, o_v0, o_v1, sc_, so0, so1, sr):
    w = lax.axis_index("s") * _NC + lax.axis_index("c")
    g = w // _NS
    k = w % _NS
    f_base = g * _FH
    d0 = 2 * k
    o_v = (o_v0, o_v1)
    so = (so0, so1)

    pltpu.async_copy(xt_hbm.at[f_base], col_v, sc_)
    pltpu.async_copy(tabs_hbm.at[f_base, d0], row_v, sr)

    def per_field(j, carry):
        f = f_base + j
        pltpu.make_async_copy(xt_hbm.at[f], col_v, sc_).wait()
        for dd in range(2):
            pltpu.make_async_copy(tabs_hbm.at[f, d0 + dd], row_v, sr).wait()
            row = f * _DIM + d0 + dd
            out_wait = [None, None]
            for c in range(_NBC):
                p = c % 2
                if out_wait[p] is not None:
                    out_wait[p].wait()
                _gather_chunk(row_v, col_v, o_v[p], c)
                out_wait[p] = pltpu.async_copy(
                    o_v[p], out_hbm.at[row, pl.ds(c * _BC, _BC)], so[p])
            if dd == 0:
                pltpu.async_copy(tabs_hbm.at[f, d0 + 1], row_v, sr)
            else:
                @pl.when(j < _FH - 1)
                def _():
                    pltpu.async_copy(xt_hbm.at[f + 1], col_v, sc_)
                    pltpu.async_copy(tabs_hbm.at[f + 1, d0], row_v, sr)
            out_wait[0].wait()
            out_wait[1].wait()
        return carry

    lax.fori_loop(0, _FH, per_field, 0)


_emb_lookup = functools.partial(
    pl.kernel,
    out_type=jax.ShapeDtypeStruct((_N_CAT * _DIM, _B), jnp.float32),
    mesh=plsc.VectorSubcoreMesh(core_axis_name="c", subcore_axis_name="s"),
    scratch_types=[
        pltpu.VMEM((_V,), jnp.float32),
        pltpu.VMEM((_B,), jnp.float32),
        pltpu.VMEM((_BC,), jnp.float32),
        pltpu.VMEM((_BC,), jnp.float32),
        pltpu.SemaphoreType.DMA,
        pltpu.SemaphoreType.DMA,
        pltpu.SemaphoreType.DMA,
        pltpu.SemaphoreType.DMA,
    ],
    compiler_params=pltpu.CompilerParams(needs_layout_passes=False),
)(_emb_body)


def kernel(X, tables):
    tabs_t = tables.transpose(0, 2, 1)
    xt = X.T
    out_t = _emb_lookup(tabs_t, xt)
    x_emb = out_t.T
    x_cont = X[:, _N_CAT:]
    return (x_emb, x_cont)


# speedup vs baseline: 1.0065x; 1.0065x over previous
import functools
import jax
import jax.numpy as jnp
from jax import lax
from jax.experimental import pallas as pl
from jax.experimental.pallas import tpu as pltpu
from jax.experimental.pallas import tpu_sc as plsc

_N_CAT = 26
_N_CONT = 13
_VOCAB = 100000
_DIM = 32
_B = 16384

_NC = 2
_NS = 16
_NW = _NC * _NS
_V = _VOCAB + 1
_BC = 4096
_NBC = _B // _BC
_L = 16
_FH = _N_CAT // 2


def _gather_chunk(row_v, col_v, ov, c):
    @plsc.parallel_loop(0, _BC, _L, unroll=8)
    def _(i):
        v = col_v[pl.ds(c * _BC + i, _L)].astype(jnp.int32)
        ov[pl.ds(i, _L)] = plsc.load_gather(row_v, [v])


def _emb_body(tabs_hbm, xt_hbm, out_hbm,
              row_v, col_v, o_v0, o_v1, sc_, so0, so1, sr):
    w = lax.axis_index("s") * _NC + lax.axis_index("c")
    g = w // _NS
    k = w % _NS
    f_base = g * _FH
    d0 = 2 * k
    o_v = (o_v0, o_v1)
    so = (so0, so1)

    pltpu.async_copy(xt_hbm.at[f_base], col_v, sc_)
    pltpu.async_copy(tabs_hbm.at[f_base, d0], row_v, sr)

    def per_field(j, carry):
        f = f_base + j
        pltpu.make_async_copy(xt_hbm.at[f], col_v, sc_).wait()
        for dd in range(2):
            pltpu.make_async_copy(tabs_hbm.at[f, d0 + dd], row_v, sr).wait()
            row = f * _DIM + d0 + dd
            out_wait = [None, None]
            for c in range(_NBC):
                p = c % 2
                if out_wait[p] is not None:
                    out_wait[p].wait()
                _gather_chunk(row_v, col_v, o_v[p], c)
                out_wait[p] = pltpu.async_copy(
                    o_v[p], out_hbm.at[row, pl.ds(c * _BC, _BC)], so[p])
            if dd == 0:
                pltpu.async_copy(tabs_hbm.at[f, d0 + 1], row_v, sr)
            else:
                @pl.when(j < _FH - 1)
                def _():
                    pltpu.async_copy(xt_hbm.at[f + 1], col_v, sc_)
                    pltpu.async_copy(tabs_hbm.at[f + 1, d0], row_v, sr)
            out_wait[0].wait()
            out_wait[1].wait()
        return carry

    lax.fori_loop(0, _FH, per_field, 0)


_emb_lookup = functools.partial(
    pl.kernel,
    out_type=jax.ShapeDtypeStruct((_N_CAT * _DIM, _B), jnp.float32),
    mesh=plsc.VectorSubcoreMesh(core_axis_name="c", subcore_axis_name="s"),
    scratch_types=[
        pltpu.VMEM((_V,), jnp.float32),
        pltpu.VMEM((_B,), jnp.float32),
        pltpu.VMEM((_BC,), jnp.float32),
        pltpu.VMEM((_BC,), jnp.float32),
        pltpu.SemaphoreType.DMA,
        pltpu.SemaphoreType.DMA,
        pltpu.SemaphoreType.DMA,
        pltpu.SemaphoreType.DMA,
    ],
    compiler_params=pltpu.CompilerParams(needs_layout_passes=False),
)(_emb_body)


def kernel(X, tables):
    tabs_t = tables.transpose(0, 2, 1)
    xt = X.T
    out_t = _emb_lookup(tabs_t, xt)
    x_emb = out_t.T
    x_cont = X[:, _N_CAT:]
    return (x_emb, x_cont)
